# baseline (speedup 1.0000x reference)
import functools

import jax
import jax.numpy as jnp
from jax import lax
from jax.experimental import pallas as pl
from jax.experimental.pallas import tpu as pltpu
from jax.experimental.pallas import tpu_sc as plsc

N_U, N_I = 10000, 1000
D_IN, D_AGG, D_OUT = 256, 256, 64
N_R = 5
E = 160000
L_IMP = 50

NC, NS = 2, 16
E_PAD = 163840
EPT = E_PAD // NS
N_UP = 10240
UPW = N_UP // (NC * NS)
DUMMY_U = 10200
DUMMY_I = 1020
DUMMY_B = 5020

_mesh = plsc.VectorSubcoreMesh(core_axis_name="c", subcore_axis_name="s")


def _fill2d(ref, nrows, ncols, value):
    v = jnp.full((16,), value, jnp.float32)

    def body(j, _):
        for k in range(ncols // 16):
            ref[j, pl.ds(k * 16, 16)] = v
        return 0

    lax.fori_loop(0, nrows, body, 0)


@functools.partial(
    pl.kernel,
    out_type=(
        jax.ShapeDtypeStruct((1280, 128), jnp.float32),
        jax.ShapeDtypeStruct((128, 128), jnp.float32),
        jax.ShapeDtypeStruct((N_UP, 128), jnp.float32),
    ),
    mesh=_mesh,
    scratch_types=[
        pltpu.VMEM((80, 128), jnp.int32),
        pltpu.VMEM((80, 128), jnp.int32),
        pltpu.VMEM((125, 128), jnp.int32),
        pltpu.VMEM((125, 128), jnp.int32),
        pltpu.VMEM((128, 128), jnp.float32),
        pltpu.VMEM((64, 128), jnp.float32),
        pltpu.VMEM_SHARED((1280, 128), jnp.float32),
        pltpu.VMEM_SHARED((N_UP // 2, 128), jnp.float32),
    ],
)
def _sc_pre(degg_hbm, degs_hbm, onest_hbm, impg_hbm, impu_hbm, yt_hbm,
            degu_out, degi_out, y_out,
            dgi_v, dsi_v, gidx_v, uidx_v, rows_v, zb_v,
            deg_sh, y_sh):
    c = lax.axis_index("c")
    s = lax.axis_index("s")
    w = c * NS + s
    _fill2d(zb_v, 64, 128, 0.0)
    pltpu.sync_copy(zb_v, deg_sh.at[pl.ds(s * 80, 64)])
    pltpu.sync_copy(zb_v.at[pl.ds(0, 16)], deg_sh.at[pl.ds(s * 80 + 64, 16)])
    for k in range(5):
        pltpu.sync_copy(zb_v, y_sh.at[pl.ds(s * 320 + k * 64, 64)])
    plsc.subcore_barrier()

    pltpu.sync_copy(degg_hbm.at[c, s], dgi_v)
    pltpu.sync_copy(degs_hbm.at[c, s], dsi_v)

    def deg_body(j, _):
        pltpu.sync_copy(onest_hbm.at[dgi_v.at[j]], rows_v)
        pltpu.sync_copy(rows_v, deg_sh.at[dsi_v.at[j]], add=True)
        return 0

    lax.fori_loop(0, 80, deg_body, 0)

    pltpu.sync_copy(impg_hbm.at[w], gidx_v)
    pltpu.sync_copy(impu_hbm.at[w], uidx_v)

    def y_body(j, _):
        pltpu.sync_copy(yt_hbm.at[gidx_v.at[j]], rows_v)
        pltpu.sync_copy(rows_v, y_sh.at[uidx_v.at[j]], add=True)
        return 0

    lax.fori_loop(0, 125, y_body, 0)
    plsc.subcore_barrier()

    @pl.when(c == 0)
    def _():
        pltpu.sync_copy(deg_sh.at[pl.ds(s * 80, 80)],
                        degu_out.at[pl.ds(s * 80, 80)])

    @pl.when(c == 1)
    def _():
        pltpu.sync_copy(deg_sh.at[pl.ds(s * 8, 8)],
                        degi_out.at[pl.ds(s * 8, 8)])

    pltpu.sync_copy(y_sh.at[pl.ds(s * 320, 320)],
                    y_out.at[pl.ds(c * 5120 + s * 320, 320)])


@functools.partial(
    pl.kernel,
    out_type=jax.ShapeDtypeStruct((2, N_UP, 128), jnp.float32),
    mesh=_mesh,
    scratch_types=[
        pltpu.VMEM((80, 128), jnp.int32),
        pltpu.VMEM((80, 128), jnp.int32),
        pltpu.VMEM((128, 128), jnp.float32),
        pltpu.VMEM_SHARED((N_UP, 128), jnp.float32),
    ],
)
def _sc_edge_a(ufs_hbm, g_hbm, s_hbm, a_out, gi_v, si_v, rows_v, acc_sh):
    c = lax.axis_index("c")
    s = lax.axis_index("s")
    w = c * NS + s
    _fill2d(rows_v, 128, 128, 0.0)
    for k in range(5):
        pltpu.sync_copy(rows_v, acc_sh.at[pl.ds(s * 640 + k * 128, 128)])
    plsc.subcore_barrier()
    pltpu.sync_copy(g_hbm.at[w], gi_v)
    pltpu.sync_copy(s_hbm.at[w], si_v)

    def body(j, _):
        pltpu.sync_copy(ufs_hbm.at[gi_v.at[j]], rows_v)
        pltpu.sync_copy(rows_v, acc_sh.at[si_v.at[j]], add=True)
        return 0

    lax.fori_loop(0, 80, body, 0)
    plsc.subcore_barrier()
    pltpu.sync_copy(acc_sh.at[pl.ds(s * 640, 640)],
                    a_out.at[c, pl.ds(s * 640, 640)])


@functools.partial(
    pl.kernel,
    out_type=jax.ShapeDtypeStruct((2, N_UP, 128), jnp.float32),
    mesh=_mesh,
    scratch_types=[
        pltpu.VMEM((80, 128), jnp.int32),
        pltpu.VMEM((80, 128), jnp.int32),
        pltpu.VMEM((128, 128), jnp.float32),
        pltpu.VMEM_SHARED((N_UP, 128), jnp.float32),
    ],
)
def _sc_edge_s(msA_hbm, msB_hbm, g_hbm, s_hbm, s_out,
               gi_v, si_v, rows_v, acc_sh):
    c = lax.axis_index("c")
    s = lax.axis_index("s")
    _fill2d(rows_v, 128, 128, 0.0)
    for k in range(5):
        pltpu.sync_copy(rows_v, acc_sh.at[pl.ds(s * 640 + k * 128, 128)])
    plsc.subcore_barrier()
    pltpu.sync_copy(g_hbm.at[s], gi_v)
    pltpu.sync_copy(s_hbm.at[s], si_v)

    @pl.when(c == 0)
    def _():
        def body(j, _):
            pltpu.sync_copy(msA_hbm.at[gi_v.at[j]], rows_v)
            pltpu.sync_copy(rows_v, acc_sh.at[si_v.at[j]], add=True)
            return 0

        lax.fori_loop(0, 80, body, 0)

    @pl.when(c == 1)
    def _():
        def body(j, _):
            pltpu.sync_copy(msB_hbm.at[gi_v.at[j]], rows_v)
            pltpu.sync_copy(rows_v, acc_sh.at[si_v.at[j]], add=True)
            return 0

        lax.fori_loop(0, 80, body, 0)

    plsc.subcore_barrier()
    pltpu.sync_copy(acc_sh.at[pl.ds(s * 640, 640)],
                    s_out.at[c, pl.ds(s * 640, 640)])


def _interleave2(idx):
    return jnp.stack([2 * idx, 2 * idx + 1], axis=-1).reshape(-1)


def _tc_scale_body(ufeat_ref, deg_ref, out_ref):
    r = jax.lax.rsqrt(jnp.maximum(deg_ref[...], 1.0))
    out_ref[...] = ufeat_ref[...] * r


def _tc_scale(ufeat, deg_u):
    return pl.pallas_call(
        _tc_scale_body,
        grid=(10,),
        in_specs=[
            pl.BlockSpec((1000, D_IN), lambda i: (i, 0)),
            pl.BlockSpec((1000, 1), lambda i: (i, 0)),
        ],
        out_specs=pl.BlockSpec((1000, D_IN), lambda i: (i, 0)),
        out_shape=jax.ShapeDtypeStruct((N_U, D_IN), jnp.float32),
    )(ufeat, deg_u)


def _tc_msgs_body(ifeat_ref, deg_ref, wi_ref, msA_ref, msB_ref, ifs_ref):
    @pl.when(pl.program_id(0) == 0)
    def _():
        r = jax.lax.rsqrt(jnp.maximum(deg_ref[...], 1.0))
        ifs_ref[...] = ifeat_ref[...] * r

    m = jnp.dot(ifs_ref[...], wi_ref[0],
                preferred_element_type=jnp.float32)
    msA_ref[0] = m[:, :128]
    msB_ref[0] = m[:, 128:]


def _tc_msgs(ifeat, deg_i, W_i):
    return pl.pallas_call(
        _tc_msgs_body,
        grid=(N_R,),
        in_specs=[
            pl.BlockSpec((N_I, D_IN), lambda r: (0, 0)),
            pl.BlockSpec((N_I, 1), lambda r: (0, 0)),
            pl.BlockSpec((1, D_IN, D_AGG), lambda r: (r, 0, 0)),
        ],
        out_specs=[
            pl.BlockSpec((1, N_I, 128), lambda r: (r, 0, 0)),
            pl.BlockSpec((1, N_I, 128), lambda r: (r, 0, 0)),
        ],
        out_shape=[
            jax.ShapeDtypeStruct((N_R, N_I, 128), jnp.float32),
            jax.ShapeDtypeStruct((N_R, N_I, 128), jnp.float32),
        ],
        scratch_shapes=[pltpu.VMEM((N_I, D_IN), jnp.float32)],
    )(ifeat, deg_i, W_i)


def _tc_q_body(a_ref, wu_ref, deg_ref, wio_ref, bio_ref, q_ref, acc_ref):
    @pl.when(pl.program_id(0) == 0)
    def _():
        acc_ref[...] = jnp.zeros_like(acc_ref)

    acc_ref[...] += jnp.dot(a_ref[0, 0] + a_ref[1, 0], wu_ref[0],
                            preferred_element_type=jnp.float32)

    @pl.when(pl.program_id(0) == N_R - 1)
    def _():
        r = jax.lax.rsqrt(jnp.maximum(deg_ref[...], 1.0))
        hi = r * acc_ref[...]
        hi = jnp.where(hi >= 0, hi, 0.1 * hi)
        q_ref[...] = jnp.dot(hi, wio_ref[...],
                             preferred_element_type=jnp.float32) + bio_ref[...]


def _tc_q(A5, W_u, deg_i, W_io, b_io):
    return pl.pallas_call(
        _tc_q_body,
        grid=(N_R,),
        in_specs=[
            pl.BlockSpec((2, 1, N_I, D_AGG), lambda r: (0, r, 0, 0)),
            pl.BlockSpec((1, D_IN, D_AGG), lambda r: (r, 0, 0)),
            pl.BlockSpec((N_I, 1), lambda r: (0, 0)),
            pl.BlockSpec((D_AGG, D_OUT), lambda r: (0, 0)),
            pl.BlockSpec((1, D_OUT), lambda r: (0, 0)),
        ],
        out_specs=pl.BlockSpec((N_I, D_OUT), lambda r: (0, 0)),
        out_shape=jax.ShapeDtypeStruct((N_I, D_OUT), jnp.float32),
        scratch_shapes=[pltpu.VMEM((N_I, D_AGG), jnp.float32)],
    )(A5, W_u, deg_i, W_io, b_io)


def _tc_z_body(s0_ref, s1_ref, deg_ref, wuo_ref, buo_ref, y_ref, sq_ref,
               z_ref):
    r = jax.lax.rsqrt(jnp.maximum(deg_ref[...], 1.0))
    h0 = r * s0_ref[...]
    h1 = r * s1_ref[...]
    h0 = jnp.where(h0 >= 0, h0, 0.1 * h0)
    h1 = jnp.where(h1 >= 0, h1, 0.1 * h1)
    p = (jnp.dot(h0, wuo_ref[:128], preferred_element_type=jnp.float32)
         + jnp.dot(h1, wuo_ref[128:], preferred_element_type=jnp.float32))
    z_ref[...] = p + buo_ref[...] + y_ref[...] / sq_ref[...]


def _tc_z(S0, S1, deg_u, W_uo, b_uo, y_acc, sqrt_counts):
    return pl.pallas_call(
        _tc_z_body,
        grid=(10,),
        in_specs=[
            pl.BlockSpec((1000, 128), lambda i: (i, 0)),
            pl.BlockSpec((1000, 128), lambda i: (i, 0)),
            pl.BlockSpec((1000, 1), lambda i: (i, 0)),
            pl.BlockSpec((D_AGG, D_OUT), lambda i: (0, 0)),
            pl.BlockSpec((1, D_OUT), lambda i: (0, 0)),
            pl.BlockSpec((1000, D_OUT), lambda i: (i, 0)),
            pl.BlockSpec((1000, 1), lambda i: (i, 0)),
        ],
        out_specs=pl.BlockSpec((1000, D_OUT), lambda i: (i, 0)),
        out_shape=jax.ShapeDtypeStruct((N_U, D_OUT), jnp.float32),
    )(S0, S1, deg_u, W_uo, b_uo, y_acc, sqrt_counts)


def _tc_final_body(q_ref, z_ref, bi_ref, but_ref, gm_ref, out_ref):
    out_ref[...] = (
        jax.lax.dot_general(q_ref[...], z_ref[...],
                            (((1,), (1,)), ((), ())),
                            preferred_element_type=jnp.float32)
        + bi_ref[...] + but_ref[...] + gm_ref[0, 0])


def _tc_final(q, z, Bi, BuT, gm):
    return pl.pallas_call(
        _tc_final_body,
        out_shape=jax.ShapeDtypeStruct((N_I, N_U), jnp.float32),
    )(q, z, Bi, BuT, gm)


def kernel(ufeat, ifeat, edge_src, edge_dst, edge_rating, implicit_matrix,
           sqrt_counts, global_mean, W_u, W_i, W_uo, b_uo, W_io, b_io,
           Bu, Bi, Y_table):
    pad = E_PAD - E
    src_p = jnp.concatenate([edge_src, jnp.full((pad,), DUMMY_U, jnp.int32)])
    dst_p = jnp.concatenate([edge_dst, jnp.full((pad,), DUMMY_I, jnp.int32)])
    idxb = edge_rating * N_I + edge_dst
    idxb_p = jnp.concatenate([idxb, jnp.full((pad,), DUMMY_B, jnp.int32)])
    src_g = jnp.concatenate([edge_src, jnp.zeros((pad,), jnp.int32)])
    idxb_g = jnp.concatenate([idxb, jnp.zeros((pad,), jnp.int32)])

    degg = jnp.stack([src_p % 8, dst_p % 8]).reshape(NC, NS, 80, 128)
    degs = jnp.stack([src_p // 8, dst_p // 8]).reshape(NC, NS, 80, 128)
    onest = (jnp.arange(128, dtype=jnp.int32) // 16
             == jnp.arange(8, dtype=jnp.int32)[:, None]).astype(jnp.float32)
    g0 = _interleave2(src_g).reshape(NC * NS, 80, 128)
    s0 = _interleave2(idxb_p).reshape(NC * NS, 80, 128)
    g1 = idxb_g.reshape(NS, 80, 128)
    s1 = src_p.reshape(NS, 80, 128)

    imp_p = jnp.concatenate(
        [implicit_matrix, jnp.zeros((N_UP - N_U, L_IMP), jnp.int32)])
    impg = imp_p.reshape(NC * NS, 125, 128)
    impu = jnp.repeat(jnp.arange(N_UP, dtype=jnp.int32) % 5120,
                      L_IMP).reshape(NC * NS, 125, 128)
    yt = jnp.pad(Y_table.at[0].set(0.0), ((0, 0), (0, 64)))

    degu_p, degi_p, y_pack = _sc_pre(degg, degs, onest, impg, impu, yt)
    deg_u = degu_p.reshape(1280, 8, 16)[:, :, 0].reshape(N_UP)[:N_U, None]
    deg_i = degi_p.reshape(128, 8, 16)[:, :, 0].reshape(1024)[:N_I, None]
    y_acc = y_pack[:N_U, :D_OUT]

    ufs2 = _tc_scale(ufeat, deg_u).reshape(2 * N_U, 128)
    msA5, msB5 = _tc_msgs(ifeat, deg_i, W_i)
    msA = msA5.reshape(N_R * N_I, 128)
    msB = msB5.reshape(N_R * N_I, 128)

    a_par = _sc_edge_a(ufs2, g0, s0)
    s_halves = _sc_edge_s(msA, msB, g1, s1)
    A52 = a_par.reshape(2, 5120, D_IN)[:, :N_R * N_I].reshape(
        2, N_R, N_I, D_AGG)

    q_mu = _tc_q(A52, W_u, deg_i, W_io, b_io.reshape(1, D_OUT))
    z = _tc_z(s_halves[0, :N_U], s_halves[1, :N_U], deg_u,
              W_uo, b_uo.reshape(1, D_OUT), y_acc, sqrt_counts)
    return _tc_final(q_mu, z, Bi, Bu.reshape(1, N_U),
                     global_mean.reshape(1, 1))


# speedup vs baseline: 1.0673x; 1.0673x over previous
import functools

import jax
import jax.numpy as jnp
from jax import lax
from jax.experimental import pallas as pl
from jax.experimental.pallas import tpu as pltpu
from jax.experimental.pallas import tpu_sc as plsc

N_U, N_I = 10000, 1000
D_IN, D_AGG, D_OUT = 256, 256, 64
N_R = 5
E = 160000
L_IMP = 50

NC, NS = 2, 16
E_PAD = 163840
EPT = E_PAD // NS
N_UP = 10240
UPW = N_UP // (NC * NS)
DUMMY_U = 10200
DUMMY_I = 1020
DUMMY_B = 5020

_mesh = plsc.VectorSubcoreMesh(core_axis_name="c", subcore_axis_name="s")


def _fill2d(ref, nrows, ncols, value):
    v = jnp.full((16,), value, jnp.float32)

    def body(j, _):
        for k in range(ncols // 16):
            ref[j, pl.ds(k * 16, 16)] = v
        return 0

    lax.fori_loop(0, nrows, body, 0)


def _fill3d(ref, nrows, nsub, ncols, value):
    v = jnp.full((16,), value, jnp.float32)

    def body(j, _):
        for t in range(nsub):
            for k in range(ncols // 16):
                ref[j, t, pl.ds(k * 16, 16)] = v
        return 0

    lax.fori_loop(0, nrows, body, 0)


@functools.partial(
    pl.kernel,
    out_type=(
        jax.ShapeDtypeStruct((1280, 128), jnp.float32),
        jax.ShapeDtypeStruct((128, 128), jnp.float32),
        jax.ShapeDtypeStruct((N_UP, 128), jnp.float32),
    ),
    mesh=_mesh,
    scratch_types=[
        pltpu.VMEM((80, 128), jnp.int32),
        pltpu.VMEM((80, 128), jnp.int32),
        pltpu.VMEM((125, 128), jnp.int32),
        pltpu.VMEM((125, 128), jnp.int32),
        pltpu.VMEM((128, 128), jnp.float32),
        pltpu.VMEM((64, 128), jnp.float32),
        pltpu.VMEM_SHARED((1280, 128), jnp.float32),
        pltpu.VMEM_SHARED((N_UP // 2, 128), jnp.float32),
    ],
)
def _sc_pre(degg_hbm, degs_hbm, onest_hbm, impg_hbm, impu_hbm, yt_hbm,
            degu_out, degi_out, y_out,
            dgi_v, dsi_v, gidx_v, uidx_v, rows_v, zb_v,
            deg_sh, y_sh):
    c = lax.axis_index("c")
    s = lax.axis_index("s")
    w = c * NS + s
    _fill2d(zb_v, 64, 128, 0.0)
    pltpu.sync_copy(zb_v, deg_sh.at[pl.ds(s * 80, 64)])
    pltpu.sync_copy(zb_v.at[pl.ds(0, 16)], deg_sh.at[pl.ds(s * 80 + 64, 16)])
    for k in range(5):
        pltpu.sync_copy(zb_v, y_sh.at[pl.ds(s * 320 + k * 64, 64)])
    plsc.subcore_barrier()

    pltpu.sync_copy(degg_hbm.at[c, s], dgi_v)
    pltpu.sync_copy(degs_hbm.at[c, s], dsi_v)

    def deg_body(j, _):
        pltpu.sync_copy(onest_hbm.at[dgi_v.at[j]], rows_v)
        pltpu.sync_copy(rows_v, deg_sh.at[dsi_v.at[j]], add=True)
        return 0

    lax.fori_loop(0, 80, deg_body, 0)

    pltpu.sync_copy(impg_hbm.at[w], gidx_v)
    pltpu.sync_copy(impu_hbm.at[w], uidx_v)

    def y_body(j, _):
        pltpu.sync_copy(yt_hbm.at[gidx_v.at[j]], rows_v)
        pltpu.sync_copy(rows_v, y_sh.at[uidx_v.at[j]], add=True)
        return 0

    lax.fori_loop(0, 125, y_body, 0)
    plsc.subcore_barrier()

    @pl.when(c == 0)
    def _():
        pltpu.sync_copy(deg_sh.at[pl.ds(s * 80, 80)],
                        degu_out.at[pl.ds(s * 80, 80)])

    @pl.when(c == 1)
    def _():
        pltpu.sync_copy(deg_sh.at[pl.ds(s * 8, 8)],
                        degi_out.at[pl.ds(s * 8, 8)])

    pltpu.sync_copy(y_sh.at[pl.ds(s * 320, 320)],
                    y_out.at[pl.ds(c * 5120 + s * 320, 320)])


@functools.partial(
    pl.kernel,
    out_type=jax.ShapeDtypeStruct((2, 5120, 2, 128), jnp.float32),
    mesh=_mesh,
    scratch_types=[
        pltpu.VMEM((80, 64), jnp.int32),
        pltpu.VMEM((80, 64), jnp.int32),
        pltpu.VMEM((64, 2, 128), jnp.float32),
        pltpu.VMEM_SHARED((5120, 2, 128), jnp.float32),
    ],
)
def _sc_edge_a(ufs_hbm, g_hbm, s_hbm, a_out, gi_v, si_v, rows_v, acc_sh):
    c = lax.axis_index("c")
    s = lax.axis_index("s")
    w = c * NS + s
    _fill3d(rows_v, 64, 2, 128, 0.0)
    for k in range(5):
        pltpu.sync_copy(rows_v, acc_sh.at[pl.ds(s * 320 + k * 64, 64)])
    plsc.subcore_barrier()
    pltpu.sync_copy(g_hbm.at[w], gi_v)
    pltpu.sync_copy(s_hbm.at[w], si_v)

    def body(j, _):
        pltpu.sync_copy(ufs_hbm.at[gi_v.at[j]], rows_v)
        pltpu.sync_copy(rows_v, acc_sh.at[si_v.at[j]], add=True)
        return 0

    lax.fori_loop(0, 80, body, 0)
    plsc.subcore_barrier()
    pltpu.sync_copy(acc_sh.at[pl.ds(s * 320, 320)],
                    a_out.at[c, pl.ds(s * 320, 320)])


@functools.partial(
    pl.kernel,
    out_type=jax.ShapeDtypeStruct((2, N_UP, 128), jnp.float32),
    mesh=_mesh,
    scratch_types=[
        pltpu.VMEM((80, 128), jnp.int32),
        pltpu.VMEM((80, 128), jnp.int32),
        pltpu.VMEM((128, 128), jnp.float32),
        pltpu.VMEM_SHARED((N_UP, 128), jnp.float32),
    ],
)
def _sc_edge_s(msA_hbm, msB_hbm, g_hbm, s_hbm, s_out,
               gi_v, si_v, rows_v, acc_sh):
    c = lax.axis_index("c")
    s = lax.axis_index("s")
    _fill2d(rows_v, 128, 128, 0.0)
    for k in range(5):
        pltpu.sync_copy(rows_v, acc_sh.at[pl.ds(s * 640 + k * 128, 128)])
    plsc.subcore_barrier()
    pltpu.sync_copy(g_hbm.at[s], gi_v)
    pltpu.sync_copy(s_hbm.at[s], si_v)

    @pl.when(c == 0)
    def _():
        def body(j, _):
            pltpu.sync_copy(msA_hbm.at[gi_v.at[j]], rows_v)
            pltpu.sync_copy(rows_v, acc_sh.at[si_v.at[j]], add=True)
            return 0

        lax.fori_loop(0, 80, body, 0)

    @pl.when(c == 1)
    def _():
        def body(j, _):
            pltpu.sync_copy(msB_hbm.at[gi_v.at[j]], rows_v)
            pltpu.sync_copy(rows_v, acc_sh.at[si_v.at[j]], add=True)
            return 0

        lax.fori_loop(0, 80, body, 0)

    plsc.subcore_barrier()
    pltpu.sync_copy(acc_sh.at[pl.ds(s * 640, 640)],
                    s_out.at[c, pl.ds(s * 640, 640)])


def _interleave2(idx):
    return jnp.stack([2 * idx, 2 * idx + 1], axis=-1).reshape(-1)


def _tc_scale_body(ufeat_ref, deg_ref, out_ref):
    r = jax.lax.rsqrt(jnp.maximum(deg_ref[...], 1.0))
    out_ref[...] = ufeat_ref[...] * r


def _tc_scale(ufeat, deg_u):
    return pl.pallas_call(
        _tc_scale_body,
        grid=(10,),
        in_specs=[
            pl.BlockSpec((1000, D_IN), lambda i: (i, 0)),
            pl.BlockSpec((1000, 1), lambda i: (i, 0)),
        ],
        out_specs=pl.BlockSpec((1000, D_IN), lambda i: (i, 0)),
        out_shape=jax.ShapeDtypeStruct((N_U, D_IN), jnp.float32),
    )(ufeat, deg_u)


def _tc_msgs_body(ifeat_ref, deg_ref, wi_ref, msA_ref, msB_ref, ifs_ref):
    @pl.when(pl.program_id(0) == 0)
    def _():
        r = jax.lax.rsqrt(jnp.maximum(deg_ref[...], 1.0))
        ifs_ref[...] = ifeat_ref[...] * r

    m = jnp.dot(ifs_ref[...], wi_ref[0],
                preferred_element_type=jnp.float32)
    msA_ref[0] = m[:, :128]
    msB_ref[0] = m[:, 128:]


def _tc_msgs(ifeat, deg_i, W_i):
    return pl.pallas_call(
        _tc_msgs_body,
        grid=(N_R,),
        in_specs=[
            pl.BlockSpec((N_I, D_IN), lambda r: (0, 0)),
            pl.BlockSpec((N_I, 1), lambda r: (0, 0)),
            pl.BlockSpec((1, D_IN, D_AGG), lambda r: (r, 0, 0)),
        ],
        out_specs=[
            pl.BlockSpec((1, N_I, 128), lambda r: (r, 0, 0)),
            pl.BlockSpec((1, N_I, 128), lambda r: (r, 0, 0)),
        ],
        out_shape=[
            jax.ShapeDtypeStruct((N_R, N_I, 128), jnp.float32),
            jax.ShapeDtypeStruct((N_R, N_I, 128), jnp.float32),
        ],
        scratch_shapes=[pltpu.VMEM((N_I, D_IN), jnp.float32)],
    )(ifeat, deg_i, W_i)


def _tc_q_body(a_ref, wu_ref, deg_ref, wio_ref, bio_ref, q_ref, acc_ref):
    @pl.when(pl.program_id(0) == 0)
    def _():
        acc_ref[...] = jnp.zeros_like(acc_ref)

    acc_ref[...] += jnp.dot(a_ref[0, 0] + a_ref[1, 0], wu_ref[0],
                            preferred_element_type=jnp.float32)

    @pl.when(pl.program_id(0) == N_R - 1)
    def _():
        r = jax.lax.rsqrt(jnp.maximum(deg_ref[...], 1.0))
        hi = r * acc_ref[...]
        hi = jnp.where(hi >= 0, hi, 0.1 * hi)
        q_ref[...] = jnp.dot(hi, wio_ref[...],
                             preferred_element_type=jnp.float32) + bio_ref[...]


def _tc_q(A5, W_u, deg_i, W_io, b_io):
    return pl.pallas_call(
        _tc_q_body,
        grid=(N_R,),
        in_specs=[
            pl.BlockSpec((2, 1, N_I, D_AGG), lambda r: (0, r, 0, 0)),
            pl.BlockSpec((1, D_IN, D_AGG), lambda r: (r, 0, 0)),
            pl.BlockSpec((N_I, 1), lambda r: (0, 0)),
            pl.BlockSpec((D_AGG, D_OUT), lambda r: (0, 0)),
            pl.BlockSpec((1, D_OUT), lambda r: (0, 0)),
        ],
        out_specs=pl.BlockSpec((N_I, D_OUT), lambda r: (0, 0)),
        out_shape=jax.ShapeDtypeStruct((N_I, D_OUT), jnp.float32),
        scratch_shapes=[pltpu.VMEM((N_I, D_AGG), jnp.float32)],
    )(A5, W_u, deg_i, W_io, b_io)


def _tc_z_body(s0_ref, s1_ref, deg_ref, wuo_ref, buo_ref, y_ref, sq_ref,
               z_ref):
    r = jax.lax.rsqrt(jnp.maximum(deg_ref[...], 1.0))
    h0 = r * s0_ref[...]
    h1 = r * s1_ref[...]
    h0 = jnp.where(h0 >= 0, h0, 0.1 * h0)
    h1 = jnp.where(h1 >= 0, h1, 0.1 * h1)
    p = (jnp.dot(h0, wuo_ref[:128], preferred_element_type=jnp.float32)
         + jnp.dot(h1, wuo_ref[128:], preferred_element_type=jnp.float32))
    z_ref[...] = p + buo_ref[...] + y_ref[...] / sq_ref[...]


def _tc_z(S0, S1, deg_u, W_uo, b_uo, y_acc, sqrt_counts):
    return pl.pallas_call(
        _tc_z_body,
        grid=(10,),
        in_specs=[
            pl.BlockSpec((1000, 128), lambda i: (i, 0)),
            pl.BlockSpec((1000, 128), lambda i: (i, 0)),
            pl.BlockSpec((1000, 1), lambda i: (i, 0)),
            pl.BlockSpec((D_AGG, D_OUT), lambda i: (0, 0)),
            pl.BlockSpec((1, D_OUT), lambda i: (0, 0)),
            pl.BlockSpec((1000, D_OUT), lambda i: (i, 0)),
            pl.BlockSpec((1000, 1), lambda i: (i, 0)),
        ],
        out_specs=pl.BlockSpec((1000, D_OUT), lambda i: (i, 0)),
        out_shape=jax.ShapeDtypeStruct((N_U, D_OUT), jnp.float32),
    )(S0, S1, deg_u, W_uo, b_uo, y_acc, sqrt_counts)


def _tc_final_body(q_ref, z_ref, bi_ref, but_ref, gm_ref, out_ref):
    out_ref[...] = (
        jax.lax.dot_general(q_ref[...], z_ref[...],
                            (((1,), (1,)), ((), ())),
                            preferred_element_type=jnp.float32)
        + bi_ref[...] + but_ref[...] + gm_ref[0, 0])


def _tc_final(q, z, Bi, BuT, gm):
    return pl.pallas_call(
        _tc_final_body,
        out_shape=jax.ShapeDtypeStruct((N_I, N_U), jnp.float32),
    )(q, z, Bi, BuT, gm)


def kernel(ufeat, ifeat, edge_src, edge_dst, edge_rating, implicit_matrix,
           sqrt_counts, global_mean, W_u, W_i, W_uo, b_uo, W_io, b_io,
           Bu, Bi, Y_table):
    pad = E_PAD - E
    src_p = jnp.concatenate([edge_src, jnp.full((pad,), DUMMY_U, jnp.int32)])
    dst_p = jnp.concatenate([edge_dst, jnp.full((pad,), DUMMY_I, jnp.int32)])
    idxb = edge_rating * N_I + edge_dst
    idxb_p = jnp.concatenate([idxb, jnp.full((pad,), DUMMY_B, jnp.int32)])
    src_g = jnp.concatenate([edge_src, jnp.zeros((pad,), jnp.int32)])
    idxb_g = jnp.concatenate([idxb, jnp.zeros((pad,), jnp.int32)])

    degg = jnp.stack([src_p % 8, dst_p % 8]).reshape(NC, NS, 80, 128)
    degs = jnp.stack([src_p // 8, dst_p // 8]).reshape(NC, NS, 80, 128)
    onest = (jnp.arange(128, dtype=jnp.int32) // 16
             == jnp.arange(8, dtype=jnp.int32)[:, None]).astype(jnp.float32)
    g0 = src_g.reshape(NC * NS, 80, 64)
    s0 = idxb_p.reshape(NC * NS, 80, 64)
    g1 = idxb_g.reshape(NS, 80, 128)
    s1 = src_p.reshape(NS, 80, 128)

    imp_p = jnp.concatenate(
        [implicit_matrix, jnp.zeros((N_UP - N_U, L_IMP), jnp.int32)])
    impg = imp_p.reshape(NC * NS, 125, 128)
    impu = jnp.repeat(jnp.arange(N_UP, dtype=jnp.int32) % 5120,
                      L_IMP).reshape(NC * NS, 125, 128)
    yt = jnp.pad(Y_table.at[0].set(0.0), ((0, 0), (0, 64)))

    degu_p, degi_p, y_pack = _sc_pre(degg, degs, onest, impg, impu, yt)
    deg_u = degu_p.reshape(1280, 8, 16)[:, :, 0].reshape(N_UP)[:N_U, None]
    deg_i = degi_p.reshape(128, 8, 16)[:, :, 0].reshape(1024)[:N_I, None]
    y_acc = y_pack[:N_U, :D_OUT]

    ufs3 = _tc_scale(ufeat, deg_u).reshape(N_U, 2, 128)
    msA5, msB5 = _tc_msgs(ifeat, deg_i, W_i)
    msA = msA5.reshape(N_R * N_I, 128)
    msB = msB5.reshape(N_R * N_I, 128)

    a_par = _sc_edge_a(ufs3, g0, s0)
    s_halves = _sc_edge_s(msA, msB, g1, s1)
    A52 = a_par.reshape(2, 5120, D_IN)[:, :N_R * N_I].reshape(
        2, N_R, N_I, D_AGG)

    q_mu = _tc_q(A52, W_u, deg_i, W_io, b_io.reshape(1, D_OUT))
    z = _tc_z(s_halves[0, :N_U], s_halves[1, :N_U], deg_u,
              W_uo, b_uo.reshape(1, D_OUT), y_acc, sqrt_counts)
    return _tc_final(q_mu, z, Bi, Bu.reshape(1, N_U),
                     global_mean.reshape(1, 1))
